# baseline (speedup 1.0000x reference)
import jax
import jax.numpy as jnp
from jax import lax
from jax.experimental import pallas as pl
from jax.experimental.pallas import tpu as pltpu
from jax.experimental.pallas import tpu_sc as plsc

VOCAB = 100000
K = 32
KP = 128
RPV = KP // K
VR = VOCAB // RPV
B = 16384
NF = 3
NC = 2
NS = 16
L = 16
NW = NC * NS
BPW = B // NW
HALF = BPW // 2
NGH = HALF // L
BLK = 128
NBLK = VOCAB // BLK
TAIL = VOCAB - NBLK * BLK
TPW = (NBLK + NW - 1) // NW
VBLK = BLK // RPV
TAILV = TAIL // RPV

_MESH = plsc.VectorSubcoreMesh(core_axis_name="c", subcore_axis_name="s")
_PARAMS = pltpu.CompilerParams(
    needs_layout_passes=False, use_tc_tiling_on_sc=True)


def _wid():
    return lax.axis_index("s") * NC + lax.axis_index("c")


KUNR = 4


def _transpose_block(vin, vout, nvb, iota):
    vparams = []
    for v0 in range(0, nvb * RPV, L):
        vidx = v0 + iota
        vparams.append((vidx, vidx >> 2, (vidx & (RPV - 1)) * K))

    @plsc.parallel_loop(0, K // KUNR)
    def kstep(kq):
        work = []
        for uu in range(KUNR):
            k0 = kq * KUNR + uu
            kidx = (k0 + iota) & (K - 1)
            for vidx, vr, cbase in vparams:
                work.append((plsc.load_gather(vin, [kidx, vidx]),
                             vr, cbase + kidx))
        for g, vr, c in work:
            plsc.store_scatter(vout, [vr, c], g)


def _conv_body(ft0, ft1, ft2, o0, o1, o2,
               vin0, vin1, vout0, vout1, vin2, vout2,
               si0, si1, so0, so1):
    wid = _wid()
    iota = lax.iota(jnp.int32, L)
    vins = (vin0, vin1)
    vouts = (vout0, vout1)
    sis = (si0, si1)
    sos = (so0, so1)

    def bl_of(t):
        return jnp.minimum(wid + t * NW, NBLK - 1)

    for ft, o in ((ft0, o0), (ft1, o1), (ft2, o2)):
        def in_copy(t, slot):
            return pltpu.make_async_copy(
                ft.at[:, pl.ds(bl_of(t) * BLK, BLK)], vins[slot], sis[slot])

        def out_copy(t, slot):
            return pltpu.make_async_copy(
                vouts[slot], o.at[pl.ds(bl_of(t) * VBLK, VBLK), :], sos[slot])

        in_copy(0, 0).start()

        def step(tt, carry):
            for u in range(2):
                t = 2 * tt + u
                in_copy(t, u).wait()
                @pl.when(tt < TPW // 2)
                def _(t=t, u=u):
                    in_copy(t + 1, 1 - u).start()
                @pl.when(t >= 2)
                def _(t=t, u=u):
                    out_copy(t - 2, u).wait()
                _transpose_block(vins[u], vouts[u], VBLK, iota)
                out_copy(t, u).start()
            return carry

        lax.fori_loop(0, TPW // 2, step, 0)

        t = TPW - 1
        in_copy(t, t % 2).wait()
        out_copy(t - 2, (t - 2) % 2).wait()
        _transpose_block(vins[t % 2], vouts[t % 2], VBLK, iota)
        out_copy(t, t % 2).start()

        out_copy(TPW - 2, (TPW - 2) % 2).wait()
        out_copy(TPW - 1, (TPW - 1) % 2).wait()

        @pl.when(wid == 0)
        def _(ft=ft, o=o):
            c0 = NBLK * BLK
            pltpu.sync_copy(ft.at[:, pl.ds(c0, TAIL)], vin2)
            _transpose_block(vin2, vout2, TAILV, iota)
            pltpu.sync_copy(vout2, o.at[pl.ds(NBLK * VBLK, TAILV), :])


def _gather_body(idx_hbm, v0_hbm, v1_hbm, v2_hbm, out_hbm,
                 idx0, idx1, idx2, r0, r1, r2, out_v, dummy_hbm, sem):
    wid = _wid()
    base = wid * BPW

    for i, idxb in enumerate((idx0, idx1, idx2)):
        pltpu.sync_copy(idx_hbm.at[pl.ds(i * B + base, BPW)], idxb)

    iota = lax.iota(jnp.int32, L)

    for h in range(2):
        hbase = h * HALF

        def issue(g, carry):
            for idxb, v_hbm, r in ((idx0, v0_hbm, r0),
                                   (idx1, v1_hbm, r1),
                                   (idx2, v2_hbm, r2)):
                vec = idxb[pl.ds(hbase + g * L, L)]
                vrow = vec >> 2
                for jj in range(L):
                    rr = vrow[jj]
                    pltpu.async_copy(
                        v_hbm.at[pl.ds(rr, 1), :],
                        r.at[pl.ds(g * L + jj, 1), :], sem)
            return carry

        lax.fori_loop(0, NGH, issue, 0)

        for r in (r0, r1, r2):
            pltpu.make_async_copy(dummy_hbm, r, sem).wait()

        def group(g, carry):
            rowv = g * L + iota
            vec0 = idx0[pl.ds(hbase + g * L, L)]
            vec1 = idx1[pl.ds(hbase + g * L, L)]
            vec2 = idx2[pl.ds(hbase + g * L, L)]
            base0 = (vec0 & (RPV - 1)) * K
            base1 = (vec1 & (RPV - 1)) * K
            base2 = (vec2 & (RPV - 1)) * K
            acc = jnp.zeros((L,), jnp.float32)
            for kk in range(K):
                perm = (kk + iota) & (K - 1)
                a = plsc.load_gather(r0, [rowv, base0 + perm])
                b = plsc.load_gather(r1, [rowv, base1 + perm])
                c = plsc.load_gather(r2, [rowv, base2 + perm])
                acc = acc + a * b * c
            out_v[pl.ds(hbase + g * L, L)] = acc
            return carry

        lax.fori_loop(0, NGH, group, 0)

    pltpu.sync_copy(out_v, out_hbm.at[pl.ds(base, BPW)])


@jax.jit
def kernel(indices, factor0, factor1, factor2):
    idx = indices.astype(jnp.int32).reshape(NF * B)

    convert = pl.kernel(
        _conv_body,
        out_type=(jax.ShapeDtypeStruct((VR, KP), jnp.float32),) * NF,
        mesh=_MESH,
        scratch_types=[
            pltpu.VMEM((K, BLK), jnp.float32),
            pltpu.VMEM((K, BLK), jnp.float32),
            pltpu.VMEM((VBLK, KP), jnp.float32),
            pltpu.VMEM((VBLK, KP), jnp.float32),
            pltpu.VMEM((K, TAIL), jnp.float32),
            pltpu.VMEM((TAILV, KP), jnp.float32),
            pltpu.SemaphoreType.DMA,
            pltpu.SemaphoreType.DMA,
            pltpu.SemaphoreType.DMA,
            pltpu.SemaphoreType.DMA,
        ],
        compiler_params=_PARAMS,
    )
    v0, v1, v2 = convert(factor0.T, factor1.T, factor2.T)

    gather = pl.kernel(
        _gather_body,
        out_type=jax.ShapeDtypeStruct((B,), jnp.float32),
        mesh=_MESH,
        scratch_types=[
            pltpu.VMEM((BPW,), jnp.int32),
            pltpu.VMEM((BPW,), jnp.int32),
            pltpu.VMEM((BPW,), jnp.int32),
            pltpu.VMEM((HALF, KP), jnp.float32),
            pltpu.VMEM((HALF, KP), jnp.float32),
            pltpu.VMEM((HALF, KP), jnp.float32),
            pltpu.VMEM((BPW,), jnp.float32),
            pltpu.MemorySpace.HBM((HALF, KP), jnp.float32),
            pltpu.SemaphoreType.DMA,
        ],
        compiler_params=_PARAMS,
    )
    return gather(idx, v0, v1, v2)


# speedup vs baseline: 1.1873x; 1.1873x over previous
import jax
import jax.numpy as jnp
from jax import lax
from jax.experimental import pallas as pl
from jax.experimental.pallas import tpu as pltpu
from jax.experimental.pallas import tpu_sc as plsc

VOCAB = 100000
K = 32
KP = 128
RPV = KP // K
VR = VOCAB // RPV
B = 16384
NF = 3
NC = 2
NS = 16
L = 16
NW = NC * NS
BPW = B // NW
HALF = BPW // 2
NGH = HALF // L
BLK = 128
NBLK = VOCAB // BLK
TAIL = VOCAB - NBLK * BLK
TPW = (NBLK + NW - 1) // NW
VBLK = BLK // RPV
TAILV = TAIL // RPV

_MESH = plsc.VectorSubcoreMesh(core_axis_name="c", subcore_axis_name="s")
_PARAMS = pltpu.CompilerParams(
    needs_layout_passes=False, use_tc_tiling_on_sc=True)


def _wid():
    return lax.axis_index("s") * NC + lax.axis_index("c")


KUNR = 4


def _transpose_block(vin, vout, nvb, iota):
    vparams = []
    for v0 in range(0, nvb * RPV, L):
        vidx = v0 + iota
        vparams.append((vidx, vidx >> 2, (vidx & (RPV - 1)) * K))

    def kstep(kq, carry):
        work = []
        for uu in range(KUNR):
            k0 = kq * KUNR + uu
            kidx = (k0 + iota) & (K - 1)
            for vidx, vr, cbase in vparams:
                work.append((plsc.load_gather(vin, [kidx, vidx]),
                             vr, cbase + kidx))
        for g, vr, c in work:
            plsc.store_scatter(vout, [vr, c], g)
        return carry

    lax.fori_loop(0, K // KUNR, kstep, 0)


def _conv_body(ft1, ft2, o1, o2,
               vin0, vin1, vout0, vout1, vin2, vout2,
               si0, si1, so0, so1):
    wid = _wid()
    iota = lax.iota(jnp.int32, L)
    vins = (vin0, vin1)
    vouts = (vout0, vout1)
    sis = (si0, si1)
    sos = (so0, so1)

    def bl_of(t):
        return jnp.minimum(wid + t * NW, NBLK - 1)

    for ft, o in ((ft1, o1), (ft2, o2)):
        def in_copy(t, slot):
            return pltpu.make_async_copy(
                ft.at[:, pl.ds(bl_of(t) * BLK, BLK)], vins[slot], sis[slot])

        def out_copy(t, slot):
            return pltpu.make_async_copy(
                vouts[slot], o.at[pl.ds(bl_of(t) * VBLK, VBLK), :], sos[slot])

        in_copy(0, 0).start()

        def step(tt, carry):
            for u in range(2):
                t = 2 * tt + u
                in_copy(t, u).wait()
                @pl.when(tt < TPW // 2)
                def _(t=t, u=u):
                    in_copy(t + 1, 1 - u).start()
                @pl.when(t >= 2)
                def _(t=t, u=u):
                    out_copy(t - 2, u).wait()
                _transpose_block(vins[u], vouts[u], VBLK, iota)
                out_copy(t, u).start()
            return carry

        lax.fori_loop(0, TPW // 2, step, 0)

        t = TPW - 1
        in_copy(t, t % 2).wait()
        out_copy(t - 2, (t - 2) % 2).wait()
        _transpose_block(vins[t % 2], vouts[t % 2], VBLK, iota)
        out_copy(t, t % 2).start()

        out_copy(TPW - 2, (TPW - 2) % 2).wait()
        out_copy(TPW - 1, (TPW - 1) % 2).wait()

        @pl.when(wid == 0)
        def _(ft=ft, o=o):
            c0 = NBLK * BLK
            pltpu.sync_copy(ft.at[:, pl.ds(c0, TAIL)], vin2)
            _transpose_block(vin2, vout2, TAILV, iota)
            pltpu.sync_copy(vout2, o.at[pl.ds(NBLK * VBLK, TAILV), :])


def _gather_body(idx_hbm, t0_hbm, v1_hbm, v2_hbm, out_hbm,
                 idx0, idx1, idx2, r0, r1, r2, out_v,
                 dummy0_hbm, dummy_hbm, sem):
    wid = _wid()
    base = wid * BPW

    for i, idxb in enumerate((idx0, idx1, idx2)):
        pltpu.sync_copy(idx_hbm.at[pl.ds(i * B + base, BPW)], idxb)

    iota = lax.iota(jnp.int32, L)

    for h in range(2):
        hbase = h * HALF

        def issue(g, carry):
            vec = idx0[pl.ds(hbase + g * L, L)]
            for jj in range(L):
                rr = vec[jj]
                pltpu.async_copy(
                    t0_hbm.at[pl.ds(rr, 1), pl.ds(0, K)],
                    r0.at[pl.ds(g * L + jj, 1), pl.ds(0, K)], sem)
            for idxb, v_hbm, r in ((idx1, v1_hbm, r1),
                                   (idx2, v2_hbm, r2)):
                vec = idxb[pl.ds(hbase + g * L, L)]
                vrow = vec >> 2
                for jj in range(L):
                    rr = vrow[jj]
                    pltpu.async_copy(
                        v_hbm.at[pl.ds(rr, 1), :],
                        r.at[pl.ds(g * L + jj, 1), :], sem)
            return carry

        lax.fori_loop(0, NGH, issue, 0)

        pltpu.make_async_copy(dummy0_hbm, r0, sem).wait()
        for r in (r1, r2):
            pltpu.make_async_copy(dummy_hbm, r, sem).wait()

        def group(g, carry):
            rowv = g * L + iota
            vec1 = idx1[pl.ds(hbase + g * L, L)]
            vec2 = idx2[pl.ds(hbase + g * L, L)]
            base0 = jnp.zeros((L,), jnp.int32)
            base1 = (vec1 & (RPV - 1)) * K
            base2 = (vec2 & (RPV - 1)) * K
            acc = jnp.zeros((L,), jnp.float32)
            for kk in range(K):
                perm = (kk + iota) & (K - 1)
                a = plsc.load_gather(r0, [rowv, base0 + perm])
                b = plsc.load_gather(r1, [rowv, base1 + perm])
                c = plsc.load_gather(r2, [rowv, base2 + perm])
                acc = acc + a * b * c
            out_v[pl.ds(hbase + g * L, L)] = acc
            return carry

        lax.fori_loop(0, NGH, group, 0)

    pltpu.sync_copy(out_v, out_hbm.at[pl.ds(base, BPW)])


@jax.jit
def kernel(indices, factor0, factor1, factor2):
    idx = indices.astype(jnp.int32).reshape(NF * B)

    convert = pl.kernel(
        _conv_body,
        out_type=(jax.ShapeDtypeStruct((VR, KP), jnp.float32),) * 2,
        mesh=_MESH,
        scratch_types=[
            pltpu.VMEM((K, BLK), jnp.float32),
            pltpu.VMEM((K, BLK), jnp.float32),
            pltpu.VMEM((VBLK, KP), jnp.float32),
            pltpu.VMEM((VBLK, KP), jnp.float32),
            pltpu.VMEM((K, TAIL), jnp.float32),
            pltpu.VMEM((TAILV, KP), jnp.float32),
            pltpu.SemaphoreType.DMA,
            pltpu.SemaphoreType.DMA,
            pltpu.SemaphoreType.DMA,
            pltpu.SemaphoreType.DMA,
        ],
        compiler_params=_PARAMS,
    )
    v1, v2 = convert(factor1.T, factor2.T)

    gather = pl.kernel(
        _gather_body,
        out_type=jax.ShapeDtypeStruct((B,), jnp.float32),
        mesh=_MESH,
        scratch_types=[
            pltpu.VMEM((BPW,), jnp.int32),
            pltpu.VMEM((BPW,), jnp.int32),
            pltpu.VMEM((BPW,), jnp.int32),
            pltpu.VMEM((HALF, K), jnp.float32),
            pltpu.VMEM((HALF, KP), jnp.float32),
            pltpu.VMEM((HALF, KP), jnp.float32),
            pltpu.VMEM((BPW,), jnp.float32),
            pltpu.MemorySpace.HBM((HALF, K), jnp.float32),
            pltpu.MemorySpace.HBM((HALF, KP), jnp.float32),
            pltpu.SemaphoreType.DMA,
        ],
        compiler_params=_PARAMS,
    )
    return gather(idx, factor0, v1, v2)
